# baseline (device time: 16334 ns/iter reference)
import jax
import jax.numpy as jnp
from jax import lax
from jax.experimental import pallas as pl
from jax.experimental.pallas import tpu as pltpu

N_DEV = 4
B, SQ, SKV = 2, 128, 128
H_PER = 4
DH = 64
D_MODEL = 512
HD = H_PER * DH


def kernel(x, Wq, K_ext, V_ext, Wo):
    K3 = jnp.transpose(K_ext, (0, 2, 1, 3))
    V3 = jnp.transpose(V_ext, (0, 2, 1, 3))

    def body(x_ref, wq_ref, k_ref, v_ref, wo_ref, out_ref,
             comm_ref, ctxfull_ref, send_sems, recv_sems):
        my = lax.axis_index("i")
        right = lax.rem(my + 1, N_DEV)
        opp = lax.rem(my + 2, N_DEV)
        left = lax.rem(my + 3, N_DEV)

        barrier_sem = pltpu.get_barrier_semaphore()
        for nbr in (left, right, opp):
            pl.semaphore_signal(
                barrier_sem, inc=1,
                device_id=(nbr,), device_id_type=pl.DeviceIdType.MESH,
            )

        wq_slice = wq_ref[:, pl.ds(my * HD, HD)].astype(jnp.bfloat16)

        row_blk = lax.broadcasted_iota(jnp.int32, (SQ, SKV), 0) // 64
        col_blk = lax.broadcasted_iota(jnp.int32, (SQ, SKV), 1) // 64
        mask = (col_blk <= row_blk)[None]

        xf = x_ref[...].reshape(B * SQ, D_MODEL).astype(jnp.bfloat16)
        qf = jnp.dot(xf, wq_slice,
                     preferred_element_type=jnp.float32)

        pl.semaphore_wait(barrier_sem, N_DEV - 1)

        def mk(b, target, dst_slot, i):
            return pltpu.make_async_remote_copy(
                src_ref=comm_ref.at[0, b],
                dst_ref=comm_ref.at[dst_slot, b],
                send_sem=send_sems.at[b, i],
                recv_sem=recv_sems.at[b, i],
                device_id=(target,),
                device_id_type=pl.DeviceIdType.MESH,
            )

        rdmas = [[mk(b, right, 3, 0), mk(b, left, 1, 1), mk(b, opp, 2, 2)]
                 for b in range(B)]

        dn_qkT = (((2,), (2,)), ((0,), (0,)))
        dn_wv = (((2,), (1,)), ((0,), (0,)))
        for b in range(B):
            qb = qf[b * SQ:(b + 1) * SQ]
            qs = jnp.stack([qb[:, h * DH:(h + 1) * DH] for h in range(H_PER)]
                           ).astype(jnp.bfloat16)
            ks = k_ref[b].astype(jnp.bfloat16)
            vs = v_ref[b].astype(jnp.bfloat16)
            s = lax.dot_general(qs, ks, dn_qkT,
                                preferred_element_type=jnp.float32) * 0.125
            w = jnp.exp(jnp.where(mask, s, -1e9))
            w = w / jnp.sum(w, axis=-1, keepdims=True)
            ctx = lax.dot_general(w.astype(jnp.bfloat16), vs, dn_wv,
                                  preferred_element_type=jnp.float32)
            for h in range(H_PER):
                comm_ref[0, b, :, h * DH:(h + 1) * DH] = (
                    ctx[h].astype(jnp.bfloat16))
            for r in rdmas[b]:
                r.start()

        ctxfull_ref[:, pl.ds(my * HD, HD)] = comm_ref[0].reshape(B * SQ, HD)

        for i, o in ((1, 1), (0, 3), (2, 2)):
            for b in range(B):
                rdmas[b][i].wait_recv()
            src_dev = lax.rem(my + o, N_DEV)
            ctxfull_ref[:, pl.ds(src_dev * HD, HD)] = (
                comm_ref[o].reshape(B * SQ, HD))

        out = jnp.dot(ctxfull_ref[...], wo_ref[...].astype(jnp.bfloat16),
                      preferred_element_type=jnp.float32)
        out_ref[...] = out.reshape(B, SQ, D_MODEL)

        for b in range(B):
            for r in rdmas[b]:
                r.wait_send()

    return pl.pallas_call(
        body,
        out_shape=jax.ShapeDtypeStruct((B, SQ, D_MODEL), jnp.float32),
        in_specs=[pl.BlockSpec(memory_space=pltpu.VMEM)] * 5,
        out_specs=pl.BlockSpec(memory_space=pltpu.VMEM),
        scratch_shapes=[
            pltpu.VMEM((N_DEV, B, SQ, HD), jnp.bfloat16),
            pltpu.VMEM((B * SQ, N_DEV * HD), jnp.bfloat16),
            pltpu.SemaphoreType.DMA((B, 3)),
            pltpu.SemaphoreType.DMA((B, 3)),
        ],
        compiler_params=pltpu.CompilerParams(collective_id=0),
    )(x, Wq, K3, V3, Wo)
